# baseline (speedup 1.0000x reference)
import jax
import jax.numpy as jnp
from jax.experimental import pallas as pl

_K = 192
_C = 768
_HW = 1024
_B = 32
_NS = 4
_CS = _C // _NS


def _body(a0, a1, a2, a3, o_ref):
    parts = (a0, a1, a2, a3)
    sums = jnp.concatenate([jnp.sum(p[0], axis=1) for p in parts])
    sums = jnp.where(sums == 0.0, 0.0, sums)
    bits = jax.lax.bitcast_convert_type(sums, jnp.int32)
    skey = jnp.where(bits < 0, bits ^ jnp.int32(0x7FFFFFFF), bits)

    krow = skey.reshape(1, _C)
    kcol = skey.reshape(_C, 1)
    irow = jax.lax.broadcasted_iota(jnp.int32, (1, _C), 1)
    icol = jax.lax.broadcasted_iota(jnp.int32, (_C, 1), 0)
    beats = (kcol > krow) | ((kcol == krow) & (icol < irow))
    rank = jnp.sum(jnp.where(beats, 1, 0).astype(jnp.int32),
                   axis=0, keepdims=True)
    mask = jnp.where(rank < _K, 1.0, 0.0).astype(jnp.float32)
    acc = jax.lax.dot(mask[:, 0:_CS], parts[0][0],
                      preferred_element_type=jnp.float32)
    for i in range(1, _NS):
        acc = acc + jax.lax.dot(mask[:, i * _CS:(i + 1) * _CS], parts[i][0],
                                preferred_element_type=jnp.float32)
    o_ref[0] = acc * jnp.float32(1.0 / _K)


def kernel(a):
    a3 = a.reshape(_B, _C, _HW)
    out = pl.pallas_call(
        _body,
        grid=(_B,),
        in_specs=[
            pl.BlockSpec((1, _CS, _HW), lambda b, i=i: (b, i, 0))
            for i in range(_NS)
        ],
        out_specs=pl.BlockSpec((1, 1, _HW), lambda b: (b, 0, 0)),
        out_shape=jax.ShapeDtypeStruct((_B, 1, _HW), jnp.float32),
    )(*([a3] * _NS))
    return out.reshape(_B, 1, 32, 32)


# speedup vs baseline: 1.1489x; 1.1489x over previous
import jax
import jax.numpy as jnp
from jax.experimental import pallas as pl

_K = 192
_C = 768
_HW = 1024
_B = 32
_BB = 4


def _one_batch(a):
    sums = jnp.sum(a, axis=1)
    sums = jnp.where(sums == 0.0, 0.0, sums)
    bits = jax.lax.bitcast_convert_type(sums, jnp.int32)
    skey = jnp.where(bits < 0, bits ^ jnp.int32(0x7FFFFFFF), bits)

    krow = skey.reshape(1, _C)
    kcol = skey.reshape(_C, 1)
    irow = jax.lax.broadcasted_iota(jnp.int32, (1, _C), 1)
    icol = jax.lax.broadcasted_iota(jnp.int32, (_C, 1), 0)
    beats = (kcol > krow) | ((kcol == krow) & (icol < irow))
    rank = jnp.sum(jnp.where(beats, 1, 0).astype(jnp.int32),
                   axis=0, keepdims=True)
    mask = jnp.where(rank < _K, 1.0, 0.0).astype(jnp.float32)
    out = jax.lax.dot(mask, a, preferred_element_type=jnp.float32)
    return out * jnp.float32(1.0 / _K)


def _body(a_ref, o_ref):
    for i in range(_BB):
        o_ref[i] = _one_batch(a_ref[i])


def kernel(a):
    a3 = a.reshape(_B, _C, _HW)
    out = pl.pallas_call(
        _body,
        grid=(_B // _BB,),
        in_specs=[pl.BlockSpec((_BB, _C, _HW), lambda b: (b, 0, 0))],
        out_specs=pl.BlockSpec((_BB, 1, _HW), lambda b: (b, 0, 0)),
        out_shape=jax.ShapeDtypeStruct((_B, 1, _HW), jnp.float32),
    )(a3)
    return out.reshape(_B, 1, 32, 32)
